# Initial kernel scaffold; baseline (speedup 1.0000x reference)
import jax
import jax.numpy as jnp
from jax.experimental import pallas as pl


def kernel(cosine, label):
    raise NotImplementedError("write your pallas kernel here")


# speedup vs baseline: 2.3728x; 2.3728x over previous
import functools
import math

import jax
import jax.numpy as jnp
from jax.experimental import pallas as pl
from jax.experimental.pallas import tpu as pltpu

_SCALE = 64.0
_MARGIN = 0.5
_COS_M = math.cos(_MARGIN)
_SIN_M = math.sin(_MARGIN)
_THRESH = -math.cos(_MARGIN)
_MONO = math.sin(_MARGIN) * _MARGIN
_NEG = -1e30


def _arc_kernel(lab_ref, x_ref, out_ref, m_s, s_s, t_s, *, BC, C, CB, R, B):
    r = pl.program_id(0)
    c = pl.program_id(1)

    @pl.when(c == 0)
    def _init():
        m_s[...] = jnp.full_like(m_s, _NEG)
        s_s[...] = jnp.zeros_like(s_s)
        t_s[...] = jnp.zeros_like(t_s)

    x = x_ref[...]
    lab = lab_ref[0]
    rel = lab - c * BC
    col = jax.lax.broadcasted_iota(jnp.int32, x.shape, 1)
    sub = col == rel
    hit = (rel >= 0) & (rel < BC)

    t = jnp.sum(jnp.where(sub, x, 0.0), axis=1, keepdims=True)
    tr = t * _COS_M - _SIN_M * jnp.sqrt(jnp.maximum(1.0 - t * t, 0.0))
    tr = jnp.where(t > _THRESH, tr, t - _MONO)
    tr_scaled = _SCALE * tr
    t_s[...] = jnp.where(hit, tr_scaled, t_s[...])

    xs = jnp.where(sub, tr_scaled, x * _SCALE)
    gcol = col + c * BC
    xs = jnp.where(gcol < C, xs, _NEG)

    bm = jnp.max(xs, axis=1, keepdims=True)
    m_old = m_s[...]
    m_new = jnp.maximum(m_old, bm)
    s_s[...] = s_s[...] * jnp.exp(m_old - m_new) + jnp.sum(
        jnp.exp(xs - m_new), axis=1, keepdims=True
    )
    m_s[...] = m_new

    @pl.when(c == CB - 1)
    def _finish():
        lse = jnp.log(s_s[...]) + m_s[...]
        part = jnp.sum(lse - t_s[...]).reshape(1, 1)

        @pl.when(r == 0)
        def _zero():
            out_ref[...] = jnp.zeros_like(out_ref)

        out_ref[...] += part

        @pl.when(r == R - 1)
        def _mean():
            out_ref[...] = out_ref[...] / B


def _build_call(B, C, BR, BC):
    R = B // BR
    CB = pl.cdiv(C, BC)
    return pl.pallas_call(
        functools.partial(_arc_kernel, BC=BC, C=C, CB=CB, R=R, B=B),
        grid=(R, CB),
        in_specs=[
            pl.BlockSpec((1, BR, 1), lambda r, c: (r, 0, 0)),
            pl.BlockSpec((BR, BC), lambda r, c: (r, c)),
        ],
        out_specs=pl.BlockSpec((1, 1), lambda r, c: (0, 0)),
        out_shape=jax.ShapeDtypeStruct((1, 1), jnp.float32),
        scratch_shapes=[
            pltpu.VMEM((BR, 1), jnp.float32),
            pltpu.VMEM((BR, 1), jnp.float32),
            pltpu.VMEM((BR, 1), jnp.float32),
        ],
    )


@jax.jit
def kernel(cosine, label):
    B, C = cosine.shape
    BR, BC = 256, 4096
    lab3 = label.astype(jnp.int32).reshape(B // BR, BR, 1)
    out = _build_call(B, C, BR, BC)(lab3, cosine)
    return out[0, 0]
